# baseline (speedup 1.0000x reference)
import jax
import jax.numpy as jnp
from jax import lax
from jax.experimental import pallas as pl
from jax.experimental.pallas import tpu as pltpu
from jax.experimental.pallas import tpu_sc as plsc

N = 100000
F = 128
B = 1024

NC = 2
NS = 16
NW = NC * NS
L = 16

RB = 128
MAIN = N // RB
TAIL = N - MAIN * RB
TAIL_BASE = N - RB
DUMP = B

KB = 512
NB_TC = 44
R_TC = NB_TC * KB
WIN = 128

SC_START = R_TC // RB
CPW = 18
LO_BASE = SC_START + NW * CPW
LO = MAIN - LO_BASE
CPT = CPW + 1
G = 3
SLAB = G * RB
SLOTS = CPW // G

ROWS_PER_TILE = B // NS
ZROWS = 16


def _body(feat_hbm, mem_hbm, out_hbm,
          idxs, idx1, rows0, rows1, rows2, zbuf, acc_sh,
          sem_ld0, sem_ld1, sem_lo, sem_idx, sem_add0, sem_add1):
    cid = lax.axis_index("c")
    sid = lax.axis_index("s")
    wid = cid * NS + sid
    start = SC_START + wid * CPW

    rows = (rows0, rows1)
    sem_ld = (sem_ld0, sem_ld1)
    sem_add = (sem_add0, sem_add1)

    def load_slab(b, s):
        row0 = (start + G * s) * RB
        pltpu.async_copy(feat_hbm.at[pl.ds(row0, SLAB), :], rows[b], sem_ld[b])

    def wait_load(b):
        pltpu.make_async_copy(
            feat_hbm.at[pl.ds(0, SLAB), :], rows[b], sem_ld[b]
        ).wait()

    load_slab(0, 0)

    @pl.when(wid < LO)
    def _():
        pltpu.async_copy(
            feat_hbm.at[pl.ds((LO_BASE + wid) * RB, RB), :], rows2, sem_lo
        )
        pltpu.async_copy(
            mem_hbm.at[pl.ds((LO_BASE + wid) * RB, RB)], idxs.at[CPW], sem_idx
        )

    for j in range(CPW):
        pltpu.async_copy(
            mem_hbm.at[pl.ds((start + j) * RB, RB)], idxs.at[j], sem_idx
        )

    def zero_row(r, _):
        for k in range(F // L):
            zbuf[r, pl.ds(k * L, L)] = jnp.zeros((L,), jnp.float32)
        return 0

    lax.fori_loop(0, ZROWS, zero_row, 0)

    for j in range(CPW):
        pltpu.make_async_copy(
            mem_hbm.at[pl.ds(0, RB)], idxs.at[j], sem_idx
        ).wait()

    @pl.when(wid < LO)
    def _():
        pltpu.make_async_copy(
            mem_hbm.at[pl.ds(0, RB)], idxs.at[CPW], sem_idx
        ).wait()

    for q in range(ROWS_PER_TILE // ZROWS):
        pltpu.sync_copy(
            zbuf, acc_sh.at[pl.ds(sid * ROWS_PER_TILE + q * ZROWS, ZROWS)]
        )
    plsc.subcore_barrier()

    def issue_adds(b, s):
        for g in range(G):
            pltpu.async_copy(
                rows[b].at[pl.ds(g * RB, RB)],
                acc_sh.at[idxs.at[G * s + g]],
                sem_add[b],
                add=True,
            )

    def wait_adds(b):
        for g in range(G):
            pltpu.make_async_copy(
                rows[b].at[pl.ds(g * RB, RB)],
                acc_sh.at[idxs.at[0]],
                sem_add[b],
            ).wait()

    wait_load(0)
    issue_adds(0, 0)
    load_slab(1, 1)

    def slot_pair(j2, _):
        s1 = 1 + 2 * j2
        wait_load(1)
        issue_adds(1, s1)
        wait_adds(0)
        load_slab(0, s1 + 1)
        wait_load(0)
        issue_adds(0, s1 + 1)
        wait_adds(1)
        load_slab(1, s1 + 2)
        return 0

    lax.fori_loop(0, (SLOTS - 2) // 2, slot_pair, 0)
    wait_load(1)
    issue_adds(1, SLOTS - 1)
    wait_adds(0)
    wait_adds(1)

    @pl.when(wid < LO)
    def _():
        pltpu.make_async_copy(
            feat_hbm.at[pl.ds(0, RB), :], rows2, sem_lo
        ).wait()
        pltpu.sync_copy(rows2, acc_sh.at[idxs.at[CPW]], add=True)

    @pl.when(wid == NW - 1)
    def _():
        pltpu.sync_copy(mem_hbm.at[pl.ds(TAIL_BASE, RB)], idx1)
        for t in range((RB - TAIL) // L):
            idx1[pl.ds(t * L, L)] = jnp.full((L,), DUMP, jnp.int32)
        pltpu.sync_copy(
            feat_hbm.at[pl.ds(TAIL_BASE, RB), :], rows0.at[pl.ds(0, RB)]
        )
        pltpu.sync_copy(rows0.at[pl.ds(0, RB)], acc_sh.at[idx1], add=True)

    plsc.subcore_barrier()

    r0 = sid * ROWS_PER_TILE
    pltpu.sync_copy(
        acc_sh.at[pl.ds(r0, ROWS_PER_TILE)],
        out_hbm.at[cid, pl.ds(r0, ROWS_PER_TILE), :],
    )


_segsum = pl.kernel(
    _body,
    out_type=jax.ShapeDtypeStruct((NC, B, F), jnp.float32),
    mesh=plsc.VectorSubcoreMesh(core_axis_name="c", subcore_axis_name="s"),
    scratch_types=[
        pltpu.VMEM((CPT, RB), jnp.int32),
        pltpu.VMEM((RB,), jnp.int32),
        pltpu.VMEM((SLAB, F), jnp.float32),
        pltpu.VMEM((SLAB, F), jnp.float32),
        pltpu.VMEM((RB, F), jnp.float32),
        pltpu.VMEM((ZROWS, F), jnp.float32),
        pltpu.VMEM_SHARED((B + 8, F), jnp.float32),
        pltpu.SemaphoreType.DMA,
        pltpu.SemaphoreType.DMA,
        pltpu.SemaphoreType.DMA,
        pltpu.SemaphoreType.DMA,
        pltpu.SemaphoreType.DMA,
        pltpu.SemaphoreType.DMA,
    ],
    compiler_params=pltpu.CompilerParams(use_tc_tiling_on_sc=False),
)


def _tc_body(mem_ref, rows_ref, out_ref):
    g = pl.program_id(0)

    @pl.when(g == 0)
    def _():
        out_ref[...] = jnp.zeros_like(out_ref)

    m = mem_ref[0, 0, :]
    rb = rows_ref[...].astype(jnp.bfloat16)
    m_lo = jnp.min(m)
    m_hi = jnp.max(m)
    w0 = pl.multiple_of((m_lo // 8) * 8, 8)
    narrow = (m_hi - w0) < WIN

    @pl.when(narrow)
    def _():
        ids = lax.broadcasted_iota(jnp.int32, (WIN, KB), 0) + w0
        oh = (ids == m[None, :]).astype(jnp.bfloat16)
        mm = lax.dot_general(
            oh, rb, (((1,), (0,)), ((), ())),
            preferred_element_type=jnp.float32,
        )
        out_ref[pl.ds(w0, WIN), :] += mm

    @pl.when(jnp.logical_not(narrow))
    def _():
        ids = lax.broadcasted_iota(jnp.int32, (B, KB), 0)
        oh = (ids == m[None, :]).astype(jnp.bfloat16)
        mm = lax.dot_general(
            oh, rb, (((1,), (0,)), ((), ())),
            preferred_element_type=jnp.float32,
        )
        out_ref[...] += mm


_tc_segsum = pl.pallas_call(
    _tc_body,
    grid=(NB_TC,),
    in_specs=[
        pl.BlockSpec((1, 1, KB), lambda g: (g, 0, 0)),
        pl.BlockSpec((KB, F), lambda g: (g, 0)),
    ],
    out_specs=pl.BlockSpec((B, F), lambda g: (0, 0)),
    out_shape=jax.ShapeDtypeStruct((B, F), jnp.float32),
)


@jax.jit
def kernel(atom_features, deg_slice, membership):
    del deg_slice
    m32 = membership.astype(jnp.int32)
    partials = _segsum(atom_features, m32)
    mem_tc = m32[:R_TC].reshape(NB_TC, 1, KB)
    tc_part = _tc_segsum(mem_tc, atom_features)
    return partials[0] + partials[1] + tc_part


# speedup vs baseline: 1.0958x; 1.0958x over previous
import jax
import jax.numpy as jnp
from jax import lax
from jax.experimental import pallas as pl
from jax.experimental.pallas import tpu as pltpu
from jax.experimental.pallas import tpu_sc as plsc

N = 100000
F = 128
B = 1024

NC = 2
NS = 16
NW = NC * NS
L = 16

RB = 128
MAIN = N // RB
TAIL = N - MAIN * RB
TAIL_BASE = N - RB
DUMP = B

KB = 1024
NB_TC = 22
R_TC = NB_TC * KB
WIN = 128

SC_START = R_TC // RB
CPW = 18
LO_BASE = SC_START + NW * CPW
LO = MAIN - LO_BASE
CPT = CPW + 1
G = 3
SLAB = G * RB
SLOTS = CPW // G

ROWS_PER_TILE = B // NS
ZROWS = 16


def _body(feat_hbm, mem_hbm, out_hbm,
          idxs, idx1, rows0, rows1, rows2, zbuf, acc_sh,
          sem_ld0, sem_ld1, sem_lo, sem_idx, sem_add0, sem_add1):
    cid = lax.axis_index("c")
    sid = lax.axis_index("s")
    wid = cid * NS + sid
    start = SC_START + wid * CPW

    rows = (rows0, rows1)
    sem_ld = (sem_ld0, sem_ld1)
    sem_add = (sem_add0, sem_add1)

    def load_slab(b, s):
        row0 = (start + G * s) * RB
        pltpu.async_copy(feat_hbm.at[pl.ds(row0, SLAB), :], rows[b], sem_ld[b])

    def wait_load(b):
        pltpu.make_async_copy(
            feat_hbm.at[pl.ds(0, SLAB), :], rows[b], sem_ld[b]
        ).wait()

    load_slab(0, 0)

    @pl.when(wid < LO)
    def _():
        pltpu.async_copy(
            feat_hbm.at[pl.ds((LO_BASE + wid) * RB, RB), :], rows2, sem_lo
        )
        pltpu.async_copy(
            mem_hbm.at[pl.ds((LO_BASE + wid) * RB, RB)], idxs.at[CPW], sem_idx
        )

    for j in range(CPW):
        pltpu.async_copy(
            mem_hbm.at[pl.ds((start + j) * RB, RB)], idxs.at[j], sem_idx
        )

    def zero_row(r, _):
        for k in range(F // L):
            zbuf[r, pl.ds(k * L, L)] = jnp.zeros((L,), jnp.float32)
        return 0

    lax.fori_loop(0, ZROWS, zero_row, 0)

    for j in range(CPW):
        pltpu.make_async_copy(
            mem_hbm.at[pl.ds(0, RB)], idxs.at[j], sem_idx
        ).wait()

    @pl.when(wid < LO)
    def _():
        pltpu.make_async_copy(
            mem_hbm.at[pl.ds(0, RB)], idxs.at[CPW], sem_idx
        ).wait()

    for q in range(ROWS_PER_TILE // ZROWS):
        pltpu.sync_copy(
            zbuf, acc_sh.at[pl.ds(sid * ROWS_PER_TILE + q * ZROWS, ZROWS)]
        )
    plsc.subcore_barrier()

    def issue_adds(b, s):
        for g in range(G):
            pltpu.async_copy(
                rows[b].at[pl.ds(g * RB, RB)],
                acc_sh.at[idxs.at[G * s + g]],
                sem_add[b],
                add=True,
            )

    def wait_adds(b):
        for g in range(G):
            pltpu.make_async_copy(
                rows[b].at[pl.ds(g * RB, RB)],
                acc_sh.at[idxs.at[0]],
                sem_add[b],
            ).wait()

    wait_load(0)
    issue_adds(0, 0)
    load_slab(1, 1)

    def slot_pair(j2, _):
        s1 = 1 + 2 * j2
        wait_load(1)
        issue_adds(1, s1)
        wait_adds(0)
        load_slab(0, s1 + 1)
        wait_load(0)
        issue_adds(0, s1 + 1)
        wait_adds(1)
        load_slab(1, s1 + 2)
        return 0

    lax.fori_loop(0, (SLOTS - 2) // 2, slot_pair, 0)
    wait_load(1)
    issue_adds(1, SLOTS - 1)
    wait_adds(0)
    wait_adds(1)

    @pl.when(wid < LO)
    def _():
        pltpu.make_async_copy(
            feat_hbm.at[pl.ds(0, RB), :], rows2, sem_lo
        ).wait()
        pltpu.sync_copy(rows2, acc_sh.at[idxs.at[CPW]], add=True)

    @pl.when(wid == NW - 1)
    def _():
        pltpu.sync_copy(mem_hbm.at[pl.ds(TAIL_BASE, RB)], idx1)
        for t in range((RB - TAIL) // L):
            idx1[pl.ds(t * L, L)] = jnp.full((L,), DUMP, jnp.int32)
        pltpu.sync_copy(
            feat_hbm.at[pl.ds(TAIL_BASE, RB), :], rows0.at[pl.ds(0, RB)]
        )
        pltpu.sync_copy(rows0.at[pl.ds(0, RB)], acc_sh.at[idx1], add=True)

    plsc.subcore_barrier()

    r0 = sid * ROWS_PER_TILE
    pltpu.sync_copy(
        acc_sh.at[pl.ds(r0, ROWS_PER_TILE)],
        out_hbm.at[cid, pl.ds(r0, ROWS_PER_TILE), :],
    )


_segsum = pl.kernel(
    _body,
    out_type=jax.ShapeDtypeStruct((NC, B, F), jnp.float32),
    mesh=plsc.VectorSubcoreMesh(core_axis_name="c", subcore_axis_name="s"),
    scratch_types=[
        pltpu.VMEM((CPT, RB), jnp.int32),
        pltpu.VMEM((RB,), jnp.int32),
        pltpu.VMEM((SLAB, F), jnp.float32),
        pltpu.VMEM((SLAB, F), jnp.float32),
        pltpu.VMEM((RB, F), jnp.float32),
        pltpu.VMEM((ZROWS, F), jnp.float32),
        pltpu.VMEM_SHARED((B + 8, F), jnp.float32),
        pltpu.SemaphoreType.DMA,
        pltpu.SemaphoreType.DMA,
        pltpu.SemaphoreType.DMA,
        pltpu.SemaphoreType.DMA,
        pltpu.SemaphoreType.DMA,
        pltpu.SemaphoreType.DMA,
    ],
    compiler_params=pltpu.CompilerParams(use_tc_tiling_on_sc=False),
)


def _tc_body(mem_ref, rows_ref, out_ref):
    g = pl.program_id(0)

    @pl.when(g == 0)
    def _():
        out_ref[...] = jnp.zeros_like(out_ref)

    m = mem_ref[0, 0, :]
    rb = rows_ref[...].astype(jnp.bfloat16)
    m_lo = jnp.min(m)
    m_hi = jnp.max(m)
    w0 = pl.multiple_of((m_lo // 8) * 8, 8)
    narrow = (m_hi - w0) < WIN

    @pl.when(narrow)
    def _():
        ids = lax.broadcasted_iota(jnp.int32, (WIN, KB), 0) + w0
        oh = (ids == m[None, :]).astype(jnp.bfloat16)
        mm = lax.dot_general(
            oh, rb, (((1,), (0,)), ((), ())),
            preferred_element_type=jnp.float32,
        )
        out_ref[pl.ds(w0, WIN), :] += mm

    @pl.when(jnp.logical_not(narrow))
    def _():
        ids = lax.broadcasted_iota(jnp.int32, (B, KB), 0)
        oh = (ids == m[None, :]).astype(jnp.bfloat16)
        mm = lax.dot_general(
            oh, rb, (((1,), (0,)), ((), ())),
            preferred_element_type=jnp.float32,
        )
        out_ref[...] += mm


_tc_segsum = pl.pallas_call(
    _tc_body,
    grid=(NB_TC,),
    in_specs=[
        pl.BlockSpec((1, 1, KB), lambda g: (g, 0, 0)),
        pl.BlockSpec((KB, F), lambda g: (g, 0)),
    ],
    out_specs=pl.BlockSpec((B, F), lambda g: (0, 0)),
    out_shape=jax.ShapeDtypeStruct((B, F), jnp.float32),
)


@jax.jit
def kernel(atom_features, deg_slice, membership):
    del deg_slice
    m32 = membership.astype(jnp.int32)
    partials = _segsum(atom_features, m32)
    mem_tc = m32[:R_TC].reshape(NB_TC, 1, KB)
    tc_part = _tc_segsum(mem_tc, atom_features)
    return partials[0] + partials[1] + tc_part


# speedup vs baseline: 1.1371x; 1.0377x over previous
import jax
import jax.numpy as jnp
from jax import lax
from jax.experimental import pallas as pl
from jax.experimental.pallas import tpu as pltpu
from jax.experimental.pallas import tpu_sc as plsc

N = 100000
F = 128
B = 1024

NC = 2
NS = 16
NW = NC * NS
L = 16

RB = 128
MAIN = N // RB
TAIL = N - MAIN * RB
TAIL_BASE = N - RB
DUMP = B

KB = 1024
NB_TC = 29
R_TC = NB_TC * KB
WIN = 128

SC_START = R_TC // RB
CPW = 16
LO_BASE = SC_START + NW * CPW
LO = MAIN - LO_BASE
LO2 = LO - NW
CPT = CPW + 2
G = 2
SLAB = G * RB
SLOTS = CPW // G

ROWS_PER_TILE = B // NS
ZROWS = 16


def _body(feat_hbm, mem_hbm, out_hbm,
          idxs, idx1, rows0, rows1, rows2, zbuf, acc_sh,
          sem_ld0, sem_ld1, sem_lo, sem_idx, sem_add0, sem_add1):
    cid = lax.axis_index("c")
    sid = lax.axis_index("s")
    wid = cid * NS + sid
    start = SC_START + wid * CPW

    rows = (rows0, rows1)
    sem_ld = (sem_ld0, sem_ld1)
    sem_add = (sem_add0, sem_add1)

    def load_slab(b, s):
        row0 = (start + G * s) * RB
        pltpu.async_copy(feat_hbm.at[pl.ds(row0, SLAB), :], rows[b], sem_ld[b])

    def wait_load(b):
        pltpu.make_async_copy(
            feat_hbm.at[pl.ds(0, SLAB), :], rows[b], sem_ld[b]
        ).wait()

    load_slab(0, 0)
    pltpu.async_copy(
        feat_hbm.at[pl.ds((LO_BASE + wid) * RB, RB), :], rows2, sem_lo
    )
    pltpu.async_copy(
        mem_hbm.at[pl.ds((LO_BASE + wid) * RB, RB)], idxs.at[CPW], sem_idx
    )

    @pl.when(wid < LO2)
    def _():
        pltpu.async_copy(
            mem_hbm.at[pl.ds((LO_BASE + NW + wid) * RB, RB)],
            idxs.at[CPW + 1],
            sem_idx,
        )

    for j in range(CPW):
        pltpu.async_copy(
            mem_hbm.at[pl.ds((start + j) * RB, RB)], idxs.at[j], sem_idx
        )

    def zero_row(r, _):
        for k in range(F // L):
            zbuf[r, pl.ds(k * L, L)] = jnp.zeros((L,), jnp.float32)
        return 0

    lax.fori_loop(0, ZROWS, zero_row, 0)

    for j in range(CPW):
        pltpu.make_async_copy(
            mem_hbm.at[pl.ds(0, RB)], idxs.at[j], sem_idx
        ).wait()

    pltpu.make_async_copy(
        mem_hbm.at[pl.ds(0, RB)], idxs.at[CPW], sem_idx
    ).wait()

    @pl.when(wid < LO2)
    def _():
        pltpu.make_async_copy(
            mem_hbm.at[pl.ds(0, RB)], idxs.at[CPW + 1], sem_idx
        ).wait()

    for q in range(ROWS_PER_TILE // ZROWS):
        pltpu.sync_copy(
            zbuf, acc_sh.at[pl.ds(sid * ROWS_PER_TILE + q * ZROWS, ZROWS)]
        )
    plsc.subcore_barrier()

    def issue_adds(b, s):
        for g in range(G):
            pltpu.async_copy(
                rows[b].at[pl.ds(g * RB, RB)],
                acc_sh.at[idxs.at[G * s + g]],
                sem_add[b],
                add=True,
            )

    def wait_adds(b):
        for g in range(G):
            pltpu.make_async_copy(
                rows[b].at[pl.ds(g * RB, RB)],
                acc_sh.at[idxs.at[0]],
                sem_add[b],
            ).wait()

    wait_load(0)
    issue_adds(0, 0)
    load_slab(1, 1)

    def slot_pair(j2, _):
        s1 = 1 + 2 * j2
        wait_load(1)
        issue_adds(1, s1)
        wait_adds(0)
        load_slab(0, s1 + 1)
        wait_load(0)
        issue_adds(0, s1 + 1)
        wait_adds(1)
        load_slab(1, s1 + 2)
        return 0

    lax.fori_loop(0, (SLOTS - 2) // 2, slot_pair, 0)
    wait_load(1)
    issue_adds(1, SLOTS - 1)
    wait_adds(0)
    wait_adds(1)

    pltpu.make_async_copy(feat_hbm.at[pl.ds(0, RB), :], rows2, sem_lo).wait()
    pltpu.sync_copy(rows2, acc_sh.at[idxs.at[CPW]], add=True)

    @pl.when(wid < LO2)
    def _():
        pltpu.sync_copy(
            feat_hbm.at[pl.ds((LO_BASE + NW + wid) * RB, RB), :], rows2
        )
        pltpu.sync_copy(rows2, acc_sh.at[idxs.at[CPW + 1]], add=True)

    @pl.when(wid == NW - 1)
    def _():
        pltpu.sync_copy(mem_hbm.at[pl.ds(TAIL_BASE, RB)], idx1)
        for t in range((RB - TAIL) // L):
            idx1[pl.ds(t * L, L)] = jnp.full((L,), DUMP, jnp.int32)
        pltpu.sync_copy(
            feat_hbm.at[pl.ds(TAIL_BASE, RB), :], rows0.at[pl.ds(0, RB)]
        )
        pltpu.sync_copy(rows0.at[pl.ds(0, RB)], acc_sh.at[idx1], add=True)

    plsc.subcore_barrier()

    r0 = sid * ROWS_PER_TILE
    pltpu.sync_copy(
        acc_sh.at[pl.ds(r0, ROWS_PER_TILE)],
        out_hbm.at[cid, pl.ds(r0, ROWS_PER_TILE), :],
    )


_segsum = pl.kernel(
    _body,
    out_type=jax.ShapeDtypeStruct((NC, B, F), jnp.float32),
    mesh=plsc.VectorSubcoreMesh(core_axis_name="c", subcore_axis_name="s"),
    scratch_types=[
        pltpu.VMEM((CPT, RB), jnp.int32),
        pltpu.VMEM((RB,), jnp.int32),
        pltpu.VMEM((SLAB, F), jnp.float32),
        pltpu.VMEM((SLAB, F), jnp.float32),
        pltpu.VMEM((RB, F), jnp.float32),
        pltpu.VMEM((ZROWS, F), jnp.float32),
        pltpu.VMEM_SHARED((B + 8, F), jnp.float32),
        pltpu.SemaphoreType.DMA,
        pltpu.SemaphoreType.DMA,
        pltpu.SemaphoreType.DMA,
        pltpu.SemaphoreType.DMA,
        pltpu.SemaphoreType.DMA,
        pltpu.SemaphoreType.DMA,
    ],
    compiler_params=pltpu.CompilerParams(use_tc_tiling_on_sc=False),
)


def _tc_body(mem_ref, rows_ref, out_ref):
    g = pl.program_id(0)

    @pl.when(g == 0)
    def _():
        out_ref[...] = jnp.zeros_like(out_ref)

    m = mem_ref[0, 0, :]
    rb = rows_ref[...].astype(jnp.bfloat16)
    m_lo = jnp.min(m)
    m_hi = jnp.max(m)
    w0 = pl.multiple_of((m_lo // 8) * 8, 8)
    narrow = (m_hi - w0) < WIN

    @pl.when(narrow)
    def _():
        ids = lax.broadcasted_iota(jnp.int32, (WIN, KB), 0) + w0
        oh = (ids == m[None, :]).astype(jnp.bfloat16)
        mm = lax.dot_general(
            oh, rb, (((1,), (0,)), ((), ())),
            preferred_element_type=jnp.float32,
        )
        out_ref[pl.ds(w0, WIN), :] += mm

    @pl.when(jnp.logical_not(narrow))
    def _():
        ids = lax.broadcasted_iota(jnp.int32, (B, KB), 0)
        oh = (ids == m[None, :]).astype(jnp.bfloat16)
        mm = lax.dot_general(
            oh, rb, (((1,), (0,)), ((), ())),
            preferred_element_type=jnp.float32,
        )
        out_ref[...] += mm


_tc_segsum = pl.pallas_call(
    _tc_body,
    grid=(NB_TC,),
    in_specs=[
        pl.BlockSpec((1, 1, KB), lambda g: (g, 0, 0)),
        pl.BlockSpec((KB, F), lambda g: (g, 0)),
    ],
    out_specs=pl.BlockSpec((B, F), lambda g: (0, 0)),
    out_shape=jax.ShapeDtypeStruct((B, F), jnp.float32),
)


@jax.jit
def kernel(atom_features, deg_slice, membership):
    del deg_slice
    m32 = membership.astype(jnp.int32)
    partials = _segsum(atom_features, m32)
    mem_tc = m32[:R_TC].reshape(NB_TC, 1, KB)
    tc_part = _tc_segsum(mem_tc, atom_features)
    return partials[0] + partials[1] + tc_part


# speedup vs baseline: 1.1477x; 1.0094x over previous
import jax
import jax.numpy as jnp
from jax import lax
from jax.experimental import pallas as pl
from jax.experimental.pallas import tpu as pltpu
from jax.experimental.pallas import tpu_sc as plsc

N = 100000
F = 128
B = 1024

NC = 2
NS = 16
NW = NC * NS
L = 16

RB = 128
MAIN = N // RB
TAIL = N - MAIN * RB
TAIL_BASE = N - RB
DUMP = B

KB = 1024
NB_TC = 29
R_TC = NB_TC * KB
WIN = 128

SC_START = R_TC // RB
CPW = 16
LO_BASE = SC_START + NW * CPW
LO = MAIN - LO_BASE
LO2 = LO - NW
CPT = CPW + 2
G = 2
SLAB = G * RB
SLOTS = CPW // G

ROWS_PER_TILE = B // NS
ZROWS = 16


def _body(feat_hbm, mem_hbm, out_hbm,
          idxs, idx1, rows0, rows1, rows2, zbuf, acc_sh,
          sem_ld0, sem_ld1, sem_lo, sem_idx, sem_add0, sem_add1):
    cid = lax.axis_index("c")
    sid = lax.axis_index("s")
    wid = cid * NS + sid
    start = SC_START + wid * CPW

    rows = (rows0, rows1)
    sem_ld = (sem_ld0, sem_ld1)
    sem_add = (sem_add0, sem_add1)

    def load_slab(b, s):
        row0 = (start + G * s) * RB
        pltpu.async_copy(feat_hbm.at[pl.ds(row0, SLAB), :], rows[b], sem_ld[b])

    def wait_load(b):
        pltpu.make_async_copy(
            feat_hbm.at[pl.ds(0, SLAB), :], rows[b], sem_ld[b]
        ).wait()

    load_slab(0, 0)
    pltpu.async_copy(
        feat_hbm.at[pl.ds((LO_BASE + wid) * RB, RB), :], rows2, sem_lo
    )
    pltpu.async_copy(
        mem_hbm.at[pl.ds((LO_BASE + wid) * RB, RB)], idxs.at[CPW], sem_idx
    )

    @pl.when(wid < LO2)
    def _():
        pltpu.async_copy(
            mem_hbm.at[pl.ds((LO_BASE + NW + wid) * RB, RB)],
            idxs.at[CPW + 1],
            sem_idx,
        )

    def stage_idx(j, _):
        pltpu.async_copy(
            mem_hbm.at[pl.ds((start + j) * RB, RB)], idxs.at[j], sem_idx
        )
        return 0

    lax.fori_loop(0, CPW, stage_idx, 0)

    def zero_row(r, _):
        for k in range(F // L):
            zbuf[r, pl.ds(k * L, L)] = jnp.zeros((L,), jnp.float32)
        return 0

    lax.fori_loop(0, ZROWS, zero_row, 0)

    def drain_idx(j, _):
        pltpu.make_async_copy(
            mem_hbm.at[pl.ds(0, RB)], idxs.at[j], sem_idx
        ).wait()
        return 0

    lax.fori_loop(0, CPW, drain_idx, 0)

    pltpu.make_async_copy(
        mem_hbm.at[pl.ds(0, RB)], idxs.at[CPW], sem_idx
    ).wait()

    @pl.when(wid < LO2)
    def _():
        pltpu.make_async_copy(
            mem_hbm.at[pl.ds(0, RB)], idxs.at[CPW + 1], sem_idx
        ).wait()

    for q in range(ROWS_PER_TILE // ZROWS):
        pltpu.sync_copy(
            zbuf, acc_sh.at[pl.ds(sid * ROWS_PER_TILE + q * ZROWS, ZROWS)]
        )
    plsc.subcore_barrier()

    def issue_adds(b, s):
        for g in range(G):
            pltpu.async_copy(
                rows[b].at[pl.ds(g * RB, RB)],
                acc_sh.at[idxs.at[G * s + g]],
                sem_add[b],
                add=True,
            )

    def wait_adds(b):
        for g in range(G):
            pltpu.make_async_copy(
                rows[b].at[pl.ds(g * RB, RB)],
                acc_sh.at[idxs.at[0]],
                sem_add[b],
            ).wait()

    wait_load(0)
    issue_adds(0, 0)
    load_slab(1, 1)

    def slot_pair(j2, _):
        s1 = 1 + 2 * j2
        wait_load(1)
        issue_adds(1, s1)
        wait_adds(0)
        load_slab(0, s1 + 1)
        wait_load(0)
        issue_adds(0, s1 + 1)
        wait_adds(1)
        load_slab(1, s1 + 2)
        return 0

    lax.fori_loop(0, (SLOTS - 2) // 2, slot_pair, 0)
    wait_load(1)
    issue_adds(1, SLOTS - 1)
    wait_adds(0)
    wait_adds(1)

    pltpu.make_async_copy(feat_hbm.at[pl.ds(0, RB), :], rows2, sem_lo).wait()
    pltpu.sync_copy(rows2, acc_sh.at[idxs.at[CPW]], add=True)

    @pl.when(wid < LO2)
    def _():
        pltpu.sync_copy(
            feat_hbm.at[pl.ds((LO_BASE + NW + wid) * RB, RB), :], rows2
        )
        pltpu.sync_copy(rows2, acc_sh.at[idxs.at[CPW + 1]], add=True)

    @pl.when(wid == NW - 1)
    def _():
        pltpu.sync_copy(mem_hbm.at[pl.ds(TAIL_BASE, RB)], idx1)
        for t in range((RB - TAIL) // L):
            idx1[pl.ds(t * L, L)] = jnp.full((L,), DUMP, jnp.int32)
        pltpu.sync_copy(
            feat_hbm.at[pl.ds(TAIL_BASE, RB), :], rows0.at[pl.ds(0, RB)]
        )
        pltpu.sync_copy(rows0.at[pl.ds(0, RB)], acc_sh.at[idx1], add=True)

    plsc.subcore_barrier()

    r0 = sid * ROWS_PER_TILE
    pltpu.sync_copy(
        acc_sh.at[pl.ds(r0, ROWS_PER_TILE)],
        out_hbm.at[cid, pl.ds(r0, ROWS_PER_TILE), :],
    )


_segsum = pl.kernel(
    _body,
    out_type=jax.ShapeDtypeStruct((NC, B, F), jnp.float32),
    mesh=plsc.VectorSubcoreMesh(core_axis_name="c", subcore_axis_name="s"),
    scratch_types=[
        pltpu.VMEM((CPT, RB), jnp.int32),
        pltpu.VMEM((RB,), jnp.int32),
        pltpu.VMEM((SLAB, F), jnp.float32),
        pltpu.VMEM((SLAB, F), jnp.float32),
        pltpu.VMEM((RB, F), jnp.float32),
        pltpu.VMEM((ZROWS, F), jnp.float32),
        pltpu.VMEM_SHARED((B + 8, F), jnp.float32),
        pltpu.SemaphoreType.DMA,
        pltpu.SemaphoreType.DMA,
        pltpu.SemaphoreType.DMA,
        pltpu.SemaphoreType.DMA,
        pltpu.SemaphoreType.DMA,
        pltpu.SemaphoreType.DMA,
    ],
    compiler_params=pltpu.CompilerParams(use_tc_tiling_on_sc=False),
)


def _tc_body(mem_ref, rows_ref, out_ref):
    g = pl.program_id(0)

    @pl.when(g == 0)
    def _():
        out_ref[...] = jnp.zeros_like(out_ref)

    m = mem_ref[0, 0, :]
    rb = rows_ref[...].astype(jnp.bfloat16)
    m_lo = jnp.min(m)
    m_hi = jnp.max(m)
    w0 = pl.multiple_of((m_lo // 8) * 8, 8)
    narrow = (m_hi - w0) < WIN

    @pl.when(narrow)
    def _():
        ids = lax.broadcasted_iota(jnp.int32, (WIN, KB), 0) + w0
        oh = (ids == m[None, :]).astype(jnp.bfloat16)
        mm = lax.dot_general(
            oh, rb, (((1,), (0,)), ((), ())),
            preferred_element_type=jnp.float32,
        )
        out_ref[pl.ds(w0, WIN), :] += mm

    @pl.when(jnp.logical_not(narrow))
    def _():
        ids = lax.broadcasted_iota(jnp.int32, (B, KB), 0)
        oh = (ids == m[None, :]).astype(jnp.bfloat16)
        mm = lax.dot_general(
            oh, rb, (((1,), (0,)), ((), ())),
            preferred_element_type=jnp.float32,
        )
        out_ref[...] += mm


_tc_segsum = pl.pallas_call(
    _tc_body,
    grid=(NB_TC,),
    in_specs=[
        pl.BlockSpec((1, 1, KB), lambda g: (g, 0, 0)),
        pl.BlockSpec((KB, F), lambda g: (g, 0)),
    ],
    out_specs=pl.BlockSpec((B, F), lambda g: (0, 0)),
    out_shape=jax.ShapeDtypeStruct((B, F), jnp.float32),
)


@jax.jit
def kernel(atom_features, deg_slice, membership):
    del deg_slice
    m32 = membership.astype(jnp.int32)
    partials = _segsum(atom_features, m32)
    mem_tc = m32[:R_TC].reshape(NB_TC, 1, KB)
    tc_part = _tc_segsum(mem_tc, atom_features)
    return partials[0] + partials[1] + tc_part
